# Initial kernel scaffold; baseline (speedup 1.0000x reference)
import jax
import jax.numpy as jnp
from jax.experimental import pallas as pl


def kernel(embeddings, cluster_sizes):
    raise NotImplementedError("write your pallas kernel here")


# speedup vs baseline: 8.6583x; 8.6583x over previous
import functools

import jax
import jax.numpy as jnp
from jax import lax
from jax.experimental import pallas as pl
from jax.experimental.pallas import tpu as pltpu
from jax.experimental.pallas import tpu_sc as plsc

BS, NV, DIM, NC = 8, 4096, 256, 10
NWORK = 32
RPW = NV // NWORK
NSEG = BS * NC
KCH = DIM // 16


def _sc_partial_sums(embeddings, meta):
    mesh = plsc.VectorSubcoreMesh(
        core_axis_name="c", subcore_axis_name="s",
        num_cores=2, num_subcores=16)

    @functools.partial(
        pl.kernel,
        out_type=(
            jax.ShapeDtypeStruct((NWORK, NSEG, DIM), jnp.float32),
            jax.ShapeDtypeStruct((NWORK, NSEG, DIM), jnp.float32),
        ),
        mesh=mesh,
        scratch_types=[
            pltpu.VMEM((2 * NSEG + 16,), jnp.int32),
            pltpu.VMEM((RPW, DIM), jnp.float32),
            pltpu.VMEM((NSEG, DIM), jnp.float32),
            pltpu.VMEM((NSEG, DIM), jnp.float32),
        ],
    )
    def k(emb_hbm, meta_hbm, sum_out, sq_out, meta_v, chunk_v, acc_v, sq_v):
        wid = lax.axis_index("s") * 2 + lax.axis_index("c")
        base = wid * RPW
        pltpu.sync_copy(meta_hbm, meta_v)

        for b in range(BS):
            pltpu.sync_copy(emb_hbm.at[b, pl.ds(base, RPW), :], chunk_v)

            def cluster_body(j, _, b=b):
                s = meta_v[pl.ds(b * NC + j, 16)][0]
                e = meta_v[pl.ds(NSEG + b * NC + j, 16)][0]
                lo = jnp.minimum(jnp.maximum(s - base, 0), RPW)
                hi = jnp.minimum(jnp.maximum(e - base, 0), RPW)

                def row_body(r, carry):
                    accs, sqs = carry
                    na, nq = [], []
                    for kk in range(KCH):
                        v = chunk_v[r, pl.ds(kk * 16, 16)]
                        na.append(accs[kk] + v)
                        nq.append(sqs[kk] + v * v)
                    return (tuple(na), tuple(nq))

                z = tuple(jnp.zeros((16,), jnp.float32) for _ in range(KCH))
                accs, sqs = lax.fori_loop(lo, hi, row_body, (z, z))
                seg = b * NC + j
                for kk in range(KCH):
                    acc_v[seg, pl.ds(kk * 16, 16)] = accs[kk]
                    sq_v[seg, pl.ds(kk * 16, 16)] = sqs[kk]
                return 0

            lax.fori_loop(0, NC, cluster_body, 0)

        pltpu.sync_copy(acc_v, sum_out.at[wid])
        pltpu.sync_copy(sq_v, sq_out.at[wid])

    return k(embeddings, meta)


def _tc_finish(sum_parts, sq_parts, nf):
    def body(sum_ref, sq_ref, nf_ref, out_ref):
        s = sum_ref[0]
        q = sq_ref[0]
        for w in range(1, NWORK):
            s = s + sum_ref[w]
            q = q + sq_ref[w]
        nfv = nf_ref[...]
        mean = s / nfv
        msq = jnp.sum(mean * mean, axis=1, keepdims=True)
        var_total = jnp.sum((q - nfv * mean * mean) / (nfv - 1.0))
        norm = jnp.sqrt(msq)
        m = mean / jnp.maximum(norm, 1e-12)
        g = lax.dot_general(m, m, (((1,), (1,)), ((), ())))
        row = lax.broadcasted_iota(jnp.int32, (NSEG, NSEG), 0)
        col = lax.broadcasted_iota(jnp.int32, (NSEG, NSEG), 1)
        same = ((row // NC) == (col // NC)) & (row != col)
        sum_g = jnp.sum(jnp.where(same, g, 0.0))
        pairs_per_batch = NC * (NC - 1) // 2
        loss = 0.1 * (float(BS * pairs_per_batch) + 0.5 * sum_g) + var_total
        out_ref[...] = jnp.reshape(loss, (1, 1))

    out = pl.pallas_call(
        body,
        out_shape=jax.ShapeDtypeStruct((1, 1), jnp.float32),
    )(sum_parts, sq_parts, nf)
    return out.reshape(1)


def kernel(embeddings, cluster_sizes):
    cs = cluster_sizes.astype(jnp.int32)
    starts = jnp.concatenate(
        [jnp.zeros((BS, 1), jnp.int32), cs[:, :-1]], axis=1)
    ends = starts + cs
    meta = jnp.concatenate(
        [starts.reshape(-1), ends.reshape(-1),
         jnp.zeros((16,), jnp.int32)])
    nf = jnp.broadcast_to(
        cs.astype(jnp.float32).reshape(NSEG, 1), (NSEG, DIM))
    sum_parts, sq_parts = _sc_partial_sums(embeddings, meta)
    return _tc_finish(sum_parts, sq_parts, nf)


# speedup vs baseline: 13.2787x; 1.5336x over previous
import functools

import jax
import jax.numpy as jnp
from jax import lax
from jax.experimental import pallas as pl
from jax.experimental.pallas import tpu as pltpu
from jax.experimental.pallas import tpu_sc as plsc

BS, NV, DIM, NC = 8, 4096, 256, 10
NWORK = 32
RPW = NV // NWORK
NSEG = BS * NC
KCH = DIM // 16


def _sc_partial_sums(embeddings, meta):
    mesh = plsc.VectorSubcoreMesh(
        core_axis_name="c", subcore_axis_name="s",
        num_cores=2, num_subcores=16)

    @functools.partial(
        pl.kernel,
        out_type=(
            jax.ShapeDtypeStruct((NWORK, NSEG, DIM), jnp.float32),
            jax.ShapeDtypeStruct((NWORK, NSEG, DIM), jnp.float32),
        ),
        mesh=mesh,
        scratch_types=[
            pltpu.VMEM((2 * NSEG + 16,), jnp.int32),
            pltpu.VMEM((RPW, DIM), jnp.float32),
            pltpu.VMEM((RPW, DIM), jnp.float32),
            pltpu.VMEM((NSEG, DIM), jnp.float32),
            pltpu.VMEM((NSEG, DIM), jnp.float32),
            pltpu.SemaphoreType.DMA,
            pltpu.SemaphoreType.DMA,
        ],
    )
    def k(emb_hbm, meta_hbm, sum_out, sq_out,
          meta_v, chunk0_v, chunk1_v, acc_v, sq_v, sem0, sem1):
        wid = lax.axis_index("s") * 2 + lax.axis_index("c")
        pltpu.sync_copy(meta_hbm, meta_v)
        chunks = (chunk0_v, chunk1_v)
        sems = (sem0, sem1)

        copies = [None, None]
        copies[0] = pltpu.async_copy(emb_hbm.at[0, :, wid, :], chunks[0], sems[0])
        for b in range(BS):
            cur = b % 2
            copies[cur].wait()
            if b + 1 < BS:
                nxt = (b + 1) % 2
                copies[nxt] = pltpu.async_copy(
                    emb_hbm.at[b + 1, :, wid, :], chunks[nxt], sems[nxt])
            chunk_v = chunks[cur]

            def cluster_body(j, _, b=b, chunk_v=chunk_v):
                s = meta_v[pl.ds(b * NC + j, 16)][0]
                e = meta_v[pl.ds(NSEG + b * NC + j, 16)][0]
                lo = lax.shift_right_arithmetic(s - wid + (NWORK - 1), 5)
                hi = lax.shift_right_arithmetic(e - wid + (NWORK - 1), 5)
                lo = jnp.minimum(jnp.maximum(lo, 0), RPW)
                hi = jnp.minimum(jnp.maximum(hi, 0), RPW)

                def row_body(r, carry):
                    accs, sqs = carry
                    na, nq = [], []
                    for kk in range(KCH):
                        v = chunk_v[r, pl.ds(kk * 16, 16)]
                        na.append(accs[kk] + v)
                        nq.append(sqs[kk] + v * v)
                    return (tuple(na), tuple(nq))

                z = tuple(jnp.zeros((16,), jnp.float32) for _ in range(KCH))
                accs, sqs = lax.fori_loop(lo, hi, row_body, (z, z))
                seg = b * NC + j
                for kk in range(KCH):
                    acc_v[seg, pl.ds(kk * 16, 16)] = accs[kk]
                    sq_v[seg, pl.ds(kk * 16, 16)] = sqs[kk]
                return 0

            lax.fori_loop(0, NC, cluster_body, 0)

        pltpu.sync_copy(acc_v, sum_out.at[wid])
        pltpu.sync_copy(sq_v, sq_out.at[wid])

    return k(embeddings.reshape(BS, RPW, NWORK, DIM), meta)


def _tc_finish(sum_parts, sq_parts, nf):
    def body(sum_ref, sq_ref, nf_ref, out_ref):
        s = sum_ref[0]
        q = sq_ref[0]
        for w in range(1, NWORK):
            s = s + sum_ref[w]
            q = q + sq_ref[w]
        nfv = nf_ref[...]
        mean = s / nfv
        msq = jnp.sum(mean * mean, axis=1, keepdims=True)
        var_total = jnp.sum((q - nfv * mean * mean) / (nfv - 1.0))
        norm = jnp.sqrt(msq)
        m = mean / jnp.maximum(norm, 1e-12)
        g = lax.dot_general(m, m, (((1,), (1,)), ((), ())))
        row = lax.broadcasted_iota(jnp.int32, (NSEG, NSEG), 0)
        col = lax.broadcasted_iota(jnp.int32, (NSEG, NSEG), 1)
        same = ((row // NC) == (col // NC)) & (row != col)
        sum_g = jnp.sum(jnp.where(same, g, 0.0))
        pairs_per_batch = NC * (NC - 1) // 2
        loss = 0.1 * (float(BS * pairs_per_batch) + 0.5 * sum_g) + var_total
        out_ref[...] = jnp.reshape(loss, (1, 1))

    out = pl.pallas_call(
        body,
        out_shape=jax.ShapeDtypeStruct((1, 1), jnp.float32),
    )(sum_parts, sq_parts, nf)
    return out.reshape(1)


def kernel(embeddings, cluster_sizes):
    cs = cluster_sizes.astype(jnp.int32)
    starts = jnp.concatenate(
        [jnp.zeros((BS, 1), jnp.int32), cs[:, :-1]], axis=1)
    ends = starts + cs
    meta = jnp.concatenate(
        [starts.reshape(-1), ends.reshape(-1),
         jnp.zeros((16,), jnp.int32)])
    nf = jnp.broadcast_to(
        cs.astype(jnp.float32).reshape(NSEG, 1), (NSEG, DIM))
    sum_parts, sq_parts = _sc_partial_sums(embeddings, meta)
    return _tc_finish(sum_parts, sq_parts, nf)
